# baseline (speedup 1.0000x reference)
import functools

import jax
import jax.numpy as jnp
from jax import lax
from jax.experimental import pallas as pl
from jax.experimental.pallas import tpu as pltpu
from jax.experimental.pallas import tpu_sc as plsc

N = 10000
E = 320000
D = 128
NCG = 64
NCONV = 3

NC, NS = 2, 16
NW = NC * NS
EPAD = 655360
EROWS = EPAD // 128
KR = EROWS // NW
IB = 32
NBLK = KR // IB
TROWS = 10112
ZR = TROWS // NS
DUMP = 10048
APAD = 10240
RBLK = 1000
GRID = N // RBLK

_mesh = plsc.VectorSubcoreMesh(core_axis_name="c", subcore_axis_name="s")


def _seg_sum2(xa, xb, src2d, dst2d):
  @functools.partial(
      pl.kernel,
      out_type=[jax.ShapeDtypeStruct((NC, TROWS, 64), jnp.float32),
                jax.ShapeDtypeStruct((NC, TROWS, 64), jnp.float32)],
      mesh=_mesh,
      scratch_types=[
          pltpu.VMEM((IB, 128), jnp.int32),
          pltpu.VMEM((IB, 128), jnp.int32),
          pltpu.VMEM((128, 64), jnp.float32),
          pltpu.VMEM((128, 64), jnp.float32),
          pltpu.VMEM_SHARED((TROWS, 64), jnp.float32),
          pltpu.VMEM_SHARED((TROWS, 64), jnp.float32),
          pltpu.SemaphoreType.DMA,
          pltpu.SemaphoreType.DMA,
      ],
      compiler_params=pltpu.CompilerParams(use_tc_tiling_on_sc=False),
  )
  def k(xa_hbm, xb_hbm, src_hbm, dst_hbm, zero_hbm, outa_hbm, outb_hbm,
        src_v, dst_v, rows0, rows1, tab_sh, acc_sh, semg0, semg1):
    cid = lax.axis_index("c")
    sid = lax.axis_index("s")
    wid = cid * NS + sid
    zoff = pl.multiple_of(sid * ZR, 8)
    rows = (rows0, rows1)
    semg = (semg0, semg1)

    def run_half(x_hbm, out_hbm):
      pltpu.sync_copy(x_hbm.at[pl.ds(zoff, ZR)], tab_sh.at[pl.ds(zoff, ZR)])
      pltpu.sync_copy(zero_hbm.at[pl.ds(zoff, ZR)], acc_sh.at[pl.ds(zoff, ZR)])
      plsc.subcore_barrier()

      def block(ib, carry):
        koff = pl.multiple_of(wid * KR + ib * IB, 8)
        pltpu.sync_copy(src_hbm.at[pl.ds(koff, IB)], src_v)
        pltpu.sync_copy(dst_hbm.at[pl.ds(koff, IB)], dst_v)
        pltpu.async_copy(tab_sh.at[src_v.at[0]], rows0, semg0)
        for j in range(IB):
          b = j % 2
          if j + 1 < IB:
            pltpu.async_copy(tab_sh.at[src_v.at[j + 1]], rows[1 - b],
                             semg[1 - b])
          pltpu.make_async_copy(tab_sh.at[src_v.at[j]], rows[b],
                                semg[b]).wait()
          pltpu.sync_copy(rows[b], acc_sh.at[dst_v.at[j]], add=True)
        return carry

      lax.fori_loop(0, NBLK, block, 0)
      plsc.subcore_barrier()
      pltpu.sync_copy(acc_sh.at[pl.ds(zoff, ZR)],
                      out_hbm.at[cid, pl.ds(zoff, ZR)])
      plsc.subcore_barrier()

    run_half(xa_hbm, outa_hbm)
    run_half(xb_hbm, outb_hbm)

  zero = jnp.zeros((TROWS, 64), jnp.float32)
  return k(xa, xb, src2d, dst2d, zero)


def _embed_gather(embed104, atoms2d):
  @functools.partial(
      pl.kernel,
      out_type=jax.ShapeDtypeStruct((APAD, D), jnp.float32),
      mesh=_mesh,
      scratch_types=[
          pltpu.VMEM((8, 128), jnp.int32),
          pltpu.VMEM((128, D), jnp.float32),
          pltpu.VMEM_SHARED((104, D), jnp.float32),
          pltpu.SemaphoreType.DMA,
      ],
      compiler_params=pltpu.CompilerParams(use_tc_tiling_on_sc=False),
  )
  def k(tab_hbm, atoms_hbm, out_hbm, idx_v, rows_v, tab_sh, sem):
    cid = lax.axis_index("c")
    sid = lax.axis_index("s")
    wid = cid * NS + sid

    @pl.when(sid == 0)
    def _stage():
      pltpu.sync_copy(tab_hbm, tab_sh)

    plsc.subcore_barrier()

    @pl.when(wid < APAD // 1024)
    def _work():
      pltpu.sync_copy(atoms_hbm.at[pl.ds(pl.multiple_of(wid * 8, 8), 8)],
                      idx_v)
      for j in range(8):
        pltpu.async_copy(tab_sh.at[idx_v.at[j]], rows_v, sem).wait()
        off = pl.multiple_of(wid * 1024 + j * 128, 8)
        pltpu.sync_copy(rows_v, out_hbm.at[pl.ds(off, 128)])

  return k(embed104, atoms2d)


def _mlp_first(h, W1, b1, W2, b2):
  def body(h_ref, w1_ref, b1_ref, w2_ref, b2_ref, msga_ref, msgb_ref):
    t = jnp.tanh(
        jnp.dot(h_ref[...], w1_ref[...], preferred_element_type=jnp.float32)
        + b1_ref[...])
    m = (jnp.dot(t, w2_ref[...], preferred_element_type=jnp.float32)
         + b2_ref[...])
    msga_ref[...] = m[:, :64]
    msgb_ref[...] = m[:, 64:]

  half = pl.BlockSpec((RBLK, 64), lambda i: (i, 0))
  return pl.pallas_call(
      body,
      grid=(GRID,),
      in_specs=[
          pl.BlockSpec((RBLK, D), lambda i: (i, 0)),
          pl.BlockSpec((D, D), lambda i: (0, 0)),
          pl.BlockSpec((1, D), lambda i: (0, 0)),
          pl.BlockSpec((D, D), lambda i: (0, 0)),
          pl.BlockSpec((1, D), lambda i: (0, 0)),
      ],
      out_specs=[half, half],
      out_shape=[
          jax.ShapeDtypeStruct((TROWS, 64), jnp.float32),
          jax.ShapeDtypeStruct((TROWS, 64), jnp.float32),
      ],
  )(h, W1, b1, W2, b2)


def _mlp_layer(h, pa0, pa1, pb0, pb1, W1, b1, W2, b2):
  def body(h_ref, pa0_ref, pa1_ref, pb0_ref, pb1_ref,
           w1_ref, b1_ref, w2_ref, b2_ref, hnew_ref, msga_ref, msgb_ref):
    dh = jnp.concatenate(
        [pa0_ref[...] + pa1_ref[...], pb0_ref[...] + pb1_ref[...]], axis=1)
    hn = h_ref[...] + dh
    hnew_ref[...] = hn
    t = jnp.tanh(
        jnp.dot(hn, w1_ref[...], preferred_element_type=jnp.float32)
        + b1_ref[...])
    m = (jnp.dot(t, w2_ref[...], preferred_element_type=jnp.float32)
         + b2_ref[...])
    msga_ref[...] = m[:, :64]
    msgb_ref[...] = m[:, 64:]

  half = pl.BlockSpec((RBLK, 64), lambda i: (i, 0))
  return pl.pallas_call(
      body,
      grid=(GRID,),
      in_specs=[
          pl.BlockSpec((RBLK, D), lambda i: (i, 0)),
          half, half, half, half,
          pl.BlockSpec((D, D), lambda i: (0, 0)),
          pl.BlockSpec((1, D), lambda i: (0, 0)),
          pl.BlockSpec((D, D), lambda i: (0, 0)),
          pl.BlockSpec((1, D), lambda i: (0, 0)),
      ],
      out_specs=[
          pl.BlockSpec((RBLK, D), lambda i: (i, 0)),
          half, half,
      ],
      out_shape=[
          jax.ShapeDtypeStruct((N, D), jnp.float32),
          jax.ShapeDtypeStruct((TROWS, 64), jnp.float32),
          jax.ShapeDtypeStruct((TROWS, 64), jnp.float32),
      ],
  )(h, pa0, pa1, pb0, pb1, W1, b1, W2, b2)


def _head(h, pa0, pa1, pb0, pb1, Wc1, bc1, Wc2, bc2, xyzp):
  def body(h_ref, pa0_ref, pa1_ref, pb0_ref, pb1_ref,
           w1_ref, b1_ref, w2_ref, b2_ref, xyz_ref,
           hout_ref, a_ref, colsum_ref, xyzraw_ref, cs_acc, xyz_acc):
    i = pl.program_id(0)

    @pl.when(i == 0)
    def _init():
      cs_acc[...] = jnp.zeros_like(cs_acc)
      xyz_acc[...] = jnp.zeros_like(xyz_acc)

    dh = jnp.concatenate(
        [pa0_ref[...] + pa1_ref[...], pb0_ref[...] + pb1_ref[...]], axis=1)
    hn = h_ref[...] + dh
    t = jnp.tanh(
        jnp.dot(hn, w1_ref[...], preferred_element_type=jnp.float32)
        + b1_ref[...])
    ho = (jnp.dot(t, w2_ref[...], preferred_element_type=jnp.float32)
          + b2_ref[...])
    hout_ref[...] = ho
    m = jnp.max(ho, axis=-1, keepdims=True)
    e = jnp.exp(ho - m)
    a = e / jnp.sum(e, axis=-1, keepdims=True)
    a_ref[...] = a
    cs_acc[...] += jnp.sum(a, axis=0, keepdims=True)
    xyz_acc[...] += lax.dot_general(
        a, xyz_ref[...], (((0,), (0,)), ((), ())),
        preferred_element_type=jnp.float32)

    @pl.when(i == GRID - 1)
    def _fin():
      colsum_ref[...] = cs_acc[...]
      xyzraw_ref[...] = xyz_acc[...]

  return pl.pallas_call(
      body,
      grid=(GRID,),
      in_specs=[
          pl.BlockSpec((RBLK, D), lambda i: (i, 0)),
          pl.BlockSpec((RBLK, 64), lambda i: (i, 0)),
          pl.BlockSpec((RBLK, 64), lambda i: (i, 0)),
          pl.BlockSpec((RBLK, 64), lambda i: (i, 0)),
          pl.BlockSpec((RBLK, 64), lambda i: (i, 0)),
          pl.BlockSpec((D, D), lambda i: (0, 0)),
          pl.BlockSpec((1, D), lambda i: (0, 0)),
          pl.BlockSpec((D, NCG), lambda i: (0, 0)),
          pl.BlockSpec((1, NCG), lambda i: (0, 0)),
          pl.BlockSpec((RBLK, 8), lambda i: (i, 0)),
      ],
      out_specs=[
          pl.BlockSpec((RBLK, NCG), lambda i: (i, 0)),
          pl.BlockSpec((RBLK, NCG), lambda i: (i, 0)),
          pl.BlockSpec((1, NCG), lambda i: (0, 0)),
          pl.BlockSpec((NCG, 8), lambda i: (0, 0)),
      ],
      out_shape=[
          jax.ShapeDtypeStruct((N, NCG), jnp.float32),
          jax.ShapeDtypeStruct((TROWS, NCG), jnp.float32),
          jax.ShapeDtypeStruct((1, NCG), jnp.float32),
          jax.ShapeDtypeStruct((NCG, 8), jnp.float32),
      ],
      scratch_shapes=[
          pltpu.VMEM((1, NCG), jnp.float32),
          pltpu.VMEM((NCG, 8), jnp.float32),
      ],
  )(h, pa0, pa1, pb0, pb1, Wc1, bc1, Wc2, bc2, xyzp)


def _finalize(a, g0, g1, colsum, xyzraw):
  def body(a_ref, g0_ref, g1_ref, cs_ref, xyzraw_ref,
           anorm_ref, adj_ref, cgxyz_ref, adj_acc):
    i = pl.program_id(0)

    @pl.when(i == 0)
    def _init():
      adj_acc[...] = jnp.zeros_like(adj_acc)

    rcol = 1.0 / cs_ref[...]
    a = a_ref[...]
    anorm_ref[...] = a * rcol
    adj_acc[...] += lax.dot_general(
        a, g0_ref[...] + g1_ref[...], (((0,), (0,)), ((), ())),
        preferred_element_type=jnp.float32)

    @pl.when(i == GRID - 1)
    def _fin():
      adj_ref[...] = adj_acc[...]
      r = lax.broadcasted_iota(jnp.int32, (NCG, NCG), 0)
      c = lax.broadcasted_iota(jnp.int32, (NCG, NCG), 1)
      diagm = jnp.where(r == c, jnp.broadcast_to(rcol, (NCG, NCG)), 0.0)
      cgxyz_ref[...] = jnp.dot(diagm, xyzraw_ref[...],
                               preferred_element_type=jnp.float32)

  return pl.pallas_call(
      body,
      grid=(GRID,),
      in_specs=[
          pl.BlockSpec((RBLK, NCG), lambda i: (i, 0)),
          pl.BlockSpec((RBLK, NCG), lambda i: (i, 0)),
          pl.BlockSpec((RBLK, NCG), lambda i: (i, 0)),
          pl.BlockSpec((1, NCG), lambda i: (0, 0)),
          pl.BlockSpec((NCG, 8), lambda i: (0, 0)),
      ],
      out_specs=[
          pl.BlockSpec((RBLK, NCG), lambda i: (i, 0)),
          pl.BlockSpec((NCG, NCG), lambda i: (0, 0)),
          pl.BlockSpec((NCG, 8), lambda i: (0, 0)),
      ],
      out_shape=[
          jax.ShapeDtypeStruct((N, NCG), jnp.float32),
          jax.ShapeDtypeStruct((NCG, NCG), jnp.float32),
          jax.ShapeDtypeStruct((NCG, 8), jnp.float32),
      ],
      scratch_shapes=[pltpu.VMEM((NCG, NCG), jnp.float32)],
  )(a, g0, g1, colsum, xyzraw)


def kernel(atoms_nodes, xyz, bond_edges, embed, cW1, cb1, cW2, cb2,
           Wc1, bc1, Wc2, bc2):
  bond_edges = bond_edges.astype(jnp.int32)
  atoms_nodes = atoms_nodes.astype(jnp.int32)

  e_src = jnp.concatenate([
      bond_edges[:, 1], bond_edges[:, 0],
      jnp.zeros((EPAD - 2 * E,), jnp.int32)]).reshape(EROWS, 128)
  e_dst = jnp.concatenate([
      bond_edges[:, 0], bond_edges[:, 1],
      jnp.full((EPAD - 2 * E,), DUMP, jnp.int32)]).reshape(EROWS, 128)
  atoms2d = jnp.concatenate([
      atoms_nodes, jnp.zeros((APAD - N,), jnp.int32)]).reshape(APAD // 128, 128)
  xyzp = jnp.pad(xyz, ((0, 0), (0, 5)))
  b1 = cb1.reshape(NCONV, 1, D)
  b2 = cb2.reshape(NCONV, 1, D)

  embed104 = jnp.pad(embed, ((0, 4), (0, 0)))
  h = _embed_gather(embed104, atoms2d)[:N]

  msga, msgb = _mlp_first(h, cW1[0], b1[0], cW2[0], b2[0])
  for i in range(1, NCONV):
    pa, pb = _seg_sum2(msga, msgb, e_src, e_dst)
    h, msga, msgb = _mlp_layer(h, pa[0, :N], pa[1, :N], pb[0, :N], pb[1, :N],
                               cW1[i], b1[i], cW2[i], b2[i])
  pa, pb = _seg_sum2(msga, msgb, e_src, e_dst)

  h_out, apad, colsum, xyzraw = _head(
      h, pa[0, :N], pa[1, :N], pb[0, :N], pb[1, :N],
      Wc1, bc1.reshape(1, D), Wc2, bc2.reshape(1, NCG), xyzp)
  a = apad[:N]

  ga, _gb = _seg_sum2(apad, apad, e_src, e_dst)
  anorm, cg_adj, cgxyz8 = _finalize(a, ga[0, :N], ga[1, :N], colsum, xyzraw)
  return (h_out, cgxyz8[:, :3], anorm, cg_adj)


# speedup vs baseline: 1.0610x; 1.0610x over previous
import functools

import jax
import jax.numpy as jnp
from jax import lax
from jax.experimental import pallas as pl
from jax.experimental.pallas import tpu as pltpu
from jax.experimental.pallas import tpu_sc as plsc

N = 10000
E = 320000
D = 128
NCG = 64
NCONV = 3

NC, NS = 2, 16
NW = NC * NS
EPAD = 655360
EROWS = EPAD // 128
KR = EROWS // NW
IB = 16
NBLK = KR // IB
TROWS = 10112
ZR = TROWS // NS
DUMP = 10048
APAD = 10240
RBLK = 1000
GRID = N // RBLK

_mesh = plsc.VectorSubcoreMesh(core_axis_name="c", subcore_axis_name="s")


def _seg_sum(x64, src2d, dst2d):
  @functools.partial(
      pl.kernel,
      out_type=jax.ShapeDtypeStruct((NC, TROWS, 64), jnp.float32),
      mesh=_mesh,
      scratch_types=[
          pltpu.VMEM((IB, 128), jnp.int32),
          pltpu.VMEM((IB, 128), jnp.int32),
          pltpu.VMEM((128, 64), jnp.float32),
          pltpu.VMEM((128, 64), jnp.float32),
          pltpu.VMEM_SHARED((TROWS, 64), jnp.float32),
          pltpu.VMEM_SHARED((TROWS, 64), jnp.float32),
          pltpu.SemaphoreType.DMA,
          pltpu.SemaphoreType.DMA,
      ],
      compiler_params=pltpu.CompilerParams(use_tc_tiling_on_sc=False),
  )
  def k(x_hbm, src_hbm, dst_hbm, zero_hbm, out_hbm,
        src_v, dst_v, rows0, rows1, tab_sh, acc_sh, semg0, semg1):
    cid = lax.axis_index("c")
    sid = lax.axis_index("s")
    wid = cid * NS + sid
    zoff = pl.multiple_of(sid * ZR, 8)
    pltpu.sync_copy(x_hbm.at[pl.ds(zoff, ZR)], tab_sh.at[pl.ds(zoff, ZR)])
    pltpu.sync_copy(zero_hbm.at[pl.ds(zoff, ZR)], acc_sh.at[pl.ds(zoff, ZR)])
    plsc.subcore_barrier()

    rows = (rows0, rows1)
    semg = (semg0, semg1)

    def block(ib, carry):
      koff = pl.multiple_of(wid * KR + ib * IB, 8)
      pltpu.sync_copy(src_hbm.at[pl.ds(koff, IB)], src_v)
      pltpu.sync_copy(dst_hbm.at[pl.ds(koff, IB)], dst_v)
      pltpu.async_copy(tab_sh.at[src_v.at[0]], rows0, semg0)
      for j in range(IB):
        b = j % 2
        if j + 1 < IB:
          pltpu.async_copy(tab_sh.at[src_v.at[j + 1]], rows[1 - b],
                           semg[1 - b])
        pltpu.make_async_copy(tab_sh.at[src_v.at[j]], rows[b], semg[b]).wait()
        pltpu.sync_copy(rows[b], acc_sh.at[dst_v.at[j]], add=True)
      return carry

    lax.fori_loop(0, NBLK, block, 0)
    plsc.subcore_barrier()
    pltpu.sync_copy(acc_sh.at[pl.ds(zoff, ZR)],
                    out_hbm.at[cid, pl.ds(zoff, ZR)])

  zero = jnp.zeros((TROWS, 64), jnp.float32)
  return k(x64, src2d, dst2d, zero)


def _embed_gather(embed104, atoms2d):
  @functools.partial(
      pl.kernel,
      out_type=jax.ShapeDtypeStruct((APAD, D), jnp.float32),
      mesh=_mesh,
      scratch_types=[
          pltpu.VMEM((8, 128), jnp.int32),
          pltpu.VMEM((128, D), jnp.float32),
          pltpu.VMEM_SHARED((104, D), jnp.float32),
          pltpu.SemaphoreType.DMA,
      ],
      compiler_params=pltpu.CompilerParams(use_tc_tiling_on_sc=False),
  )
  def k(tab_hbm, atoms_hbm, out_hbm, idx_v, rows_v, tab_sh, sem):
    cid = lax.axis_index("c")
    sid = lax.axis_index("s")
    wid = cid * NS + sid

    @pl.when(sid == 0)
    def _stage():
      pltpu.sync_copy(tab_hbm, tab_sh)

    plsc.subcore_barrier()

    @pl.when(wid < APAD // 1024)
    def _work():
      pltpu.sync_copy(atoms_hbm.at[pl.ds(pl.multiple_of(wid * 8, 8), 8)],
                      idx_v)
      for j in range(8):
        pltpu.async_copy(tab_sh.at[idx_v.at[j]], rows_v, sem).wait()
        off = pl.multiple_of(wid * 1024 + j * 128, 8)
        pltpu.sync_copy(rows_v, out_hbm.at[pl.ds(off, 128)])

  return k(embed104, atoms2d)


def _mlp_first(h, W1, b1, W2, b2):
  def body(h_ref, w1_ref, b1_ref, w2_ref, b2_ref, msga_ref, msgb_ref):
    t = jnp.tanh(
        jnp.dot(h_ref[...], w1_ref[...], preferred_element_type=jnp.float32)
        + b1_ref[...])
    m = (jnp.dot(t, w2_ref[...], preferred_element_type=jnp.float32)
         + b2_ref[...])
    msga_ref[...] = m[:, :64]
    msgb_ref[...] = m[:, 64:]

  half = pl.BlockSpec((RBLK, 64), lambda i: (i, 0))
  return pl.pallas_call(
      body,
      grid=(GRID,),
      in_specs=[
          pl.BlockSpec((RBLK, D), lambda i: (i, 0)),
          pl.BlockSpec((D, D), lambda i: (0, 0)),
          pl.BlockSpec((1, D), lambda i: (0, 0)),
          pl.BlockSpec((D, D), lambda i: (0, 0)),
          pl.BlockSpec((1, D), lambda i: (0, 0)),
      ],
      out_specs=[half, half],
      out_shape=[
          jax.ShapeDtypeStruct((TROWS, 64), jnp.float32),
          jax.ShapeDtypeStruct((TROWS, 64), jnp.float32),
      ],
  )(h, W1, b1, W2, b2)


def _mlp_layer(h, pa0, pa1, pb0, pb1, W1, b1, W2, b2):
  def body(h_ref, pa0_ref, pa1_ref, pb0_ref, pb1_ref,
           w1_ref, b1_ref, w2_ref, b2_ref, hnew_ref, msga_ref, msgb_ref):
    dh = jnp.concatenate(
        [pa0_ref[...] + pa1_ref[...], pb0_ref[...] + pb1_ref[...]], axis=1)
    hn = h_ref[...] + dh
    hnew_ref[...] = hn
    t = jnp.tanh(
        jnp.dot(hn, w1_ref[...], preferred_element_type=jnp.float32)
        + b1_ref[...])
    m = (jnp.dot(t, w2_ref[...], preferred_element_type=jnp.float32)
         + b2_ref[...])
    msga_ref[...] = m[:, :64]
    msgb_ref[...] = m[:, 64:]

  half = pl.BlockSpec((RBLK, 64), lambda i: (i, 0))
  return pl.pallas_call(
      body,
      grid=(GRID,),
      in_specs=[
          pl.BlockSpec((RBLK, D), lambda i: (i, 0)),
          half, half, half, half,
          pl.BlockSpec((D, D), lambda i: (0, 0)),
          pl.BlockSpec((1, D), lambda i: (0, 0)),
          pl.BlockSpec((D, D), lambda i: (0, 0)),
          pl.BlockSpec((1, D), lambda i: (0, 0)),
      ],
      out_specs=[
          pl.BlockSpec((RBLK, D), lambda i: (i, 0)),
          half, half,
      ],
      out_shape=[
          jax.ShapeDtypeStruct((N, D), jnp.float32),
          jax.ShapeDtypeStruct((TROWS, 64), jnp.float32),
          jax.ShapeDtypeStruct((TROWS, 64), jnp.float32),
      ],
  )(h, pa0, pa1, pb0, pb1, W1, b1, W2, b2)


def _head(h, pa0, pa1, pb0, pb1, Wc1, bc1, Wc2, bc2, xyzp):
  def body(h_ref, pa0_ref, pa1_ref, pb0_ref, pb1_ref,
           w1_ref, b1_ref, w2_ref, b2_ref, xyz_ref,
           hout_ref, a_ref, colsum_ref, xyzraw_ref, cs_acc, xyz_acc):
    i = pl.program_id(0)

    @pl.when(i == 0)
    def _init():
      cs_acc[...] = jnp.zeros_like(cs_acc)
      xyz_acc[...] = jnp.zeros_like(xyz_acc)

    dh = jnp.concatenate(
        [pa0_ref[...] + pa1_ref[...], pb0_ref[...] + pb1_ref[...]], axis=1)
    hn = h_ref[...] + dh
    t = jnp.tanh(
        jnp.dot(hn, w1_ref[...], preferred_element_type=jnp.float32)
        + b1_ref[...])
    ho = (jnp.dot(t, w2_ref[...], preferred_element_type=jnp.float32)
          + b2_ref[...])
    hout_ref[...] = ho
    m = jnp.max(ho, axis=-1, keepdims=True)
    e = jnp.exp(ho - m)
    a = e / jnp.sum(e, axis=-1, keepdims=True)
    a_ref[...] = a
    cs_acc[...] += jnp.sum(a, axis=0, keepdims=True)
    xyz_acc[...] += lax.dot_general(
        a, xyz_ref[...], (((0,), (0,)), ((), ())),
        preferred_element_type=jnp.float32)

    @pl.when(i == GRID - 1)
    def _fin():
      colsum_ref[...] = cs_acc[...]
      xyzraw_ref[...] = xyz_acc[...]

  return pl.pallas_call(
      body,
      grid=(GRID,),
      in_specs=[
          pl.BlockSpec((RBLK, D), lambda i: (i, 0)),
          pl.BlockSpec((RBLK, 64), lambda i: (i, 0)),
          pl.BlockSpec((RBLK, 64), lambda i: (i, 0)),
          pl.BlockSpec((RBLK, 64), lambda i: (i, 0)),
          pl.BlockSpec((RBLK, 64), lambda i: (i, 0)),
          pl.BlockSpec((D, D), lambda i: (0, 0)),
          pl.BlockSpec((1, D), lambda i: (0, 0)),
          pl.BlockSpec((D, NCG), lambda i: (0, 0)),
          pl.BlockSpec((1, NCG), lambda i: (0, 0)),
          pl.BlockSpec((RBLK, 8), lambda i: (i, 0)),
      ],
      out_specs=[
          pl.BlockSpec((RBLK, NCG), lambda i: (i, 0)),
          pl.BlockSpec((RBLK, NCG), lambda i: (i, 0)),
          pl.BlockSpec((1, NCG), lambda i: (0, 0)),
          pl.BlockSpec((NCG, 8), lambda i: (0, 0)),
      ],
      out_shape=[
          jax.ShapeDtypeStruct((N, NCG), jnp.float32),
          jax.ShapeDtypeStruct((TROWS, NCG), jnp.float32),
          jax.ShapeDtypeStruct((1, NCG), jnp.float32),
          jax.ShapeDtypeStruct((NCG, 8), jnp.float32),
      ],
      scratch_shapes=[
          pltpu.VMEM((1, NCG), jnp.float32),
          pltpu.VMEM((NCG, 8), jnp.float32),
      ],
  )(h, pa0, pa1, pb0, pb1, Wc1, bc1, Wc2, bc2, xyzp)


def _finalize(a, g0, g1, colsum, xyzraw):
  def body(a_ref, g0_ref, g1_ref, cs_ref, xyzraw_ref,
           anorm_ref, adj_ref, cgxyz_ref, adj_acc):
    i = pl.program_id(0)

    @pl.when(i == 0)
    def _init():
      adj_acc[...] = jnp.zeros_like(adj_acc)

    rcol = 1.0 / cs_ref[...]
    a = a_ref[...]
    anorm_ref[...] = a * rcol
    adj_acc[...] += lax.dot_general(
        a, g0_ref[...] + g1_ref[...], (((0,), (0,)), ((), ())),
        preferred_element_type=jnp.float32)

    @pl.when(i == GRID - 1)
    def _fin():
      adj_ref[...] = adj_acc[...]
      r = lax.broadcasted_iota(jnp.int32, (NCG, NCG), 0)
      c = lax.broadcasted_iota(jnp.int32, (NCG, NCG), 1)
      diagm = jnp.where(r == c, jnp.broadcast_to(rcol, (NCG, NCG)), 0.0)
      cgxyz_ref[...] = jnp.dot(diagm, xyzraw_ref[...],
                               preferred_element_type=jnp.float32)

  return pl.pallas_call(
      body,
      grid=(GRID,),
      in_specs=[
          pl.BlockSpec((RBLK, NCG), lambda i: (i, 0)),
          pl.BlockSpec((RBLK, NCG), lambda i: (i, 0)),
          pl.BlockSpec((RBLK, NCG), lambda i: (i, 0)),
          pl.BlockSpec((1, NCG), lambda i: (0, 0)),
          pl.BlockSpec((NCG, 8), lambda i: (0, 0)),
      ],
      out_specs=[
          pl.BlockSpec((RBLK, NCG), lambda i: (i, 0)),
          pl.BlockSpec((NCG, NCG), lambda i: (0, 0)),
          pl.BlockSpec((NCG, 8), lambda i: (0, 0)),
      ],
      out_shape=[
          jax.ShapeDtypeStruct((N, NCG), jnp.float32),
          jax.ShapeDtypeStruct((NCG, NCG), jnp.float32),
          jax.ShapeDtypeStruct((NCG, 8), jnp.float32),
      ],
      scratch_shapes=[pltpu.VMEM((NCG, NCG), jnp.float32)],
  )(a, g0, g1, colsum, xyzraw)


def kernel(atoms_nodes, xyz, bond_edges, embed, cW1, cb1, cW2, cb2,
           Wc1, bc1, Wc2, bc2):
  bond_edges = bond_edges.astype(jnp.int32)
  atoms_nodes = atoms_nodes.astype(jnp.int32)

  e_src = jnp.concatenate([
      bond_edges[:, 1], bond_edges[:, 0],
      jnp.zeros((EPAD - 2 * E,), jnp.int32)]).reshape(EROWS, 128)
  e_dst = jnp.concatenate([
      bond_edges[:, 0], bond_edges[:, 1],
      jnp.full((EPAD - 2 * E,), DUMP, jnp.int32)]).reshape(EROWS, 128)
  atoms2d = jnp.concatenate([
      atoms_nodes, jnp.zeros((APAD - N,), jnp.int32)]).reshape(APAD // 128, 128)
  xyzp = jnp.pad(xyz, ((0, 0), (0, 5)))
  b1 = cb1.reshape(NCONV, 1, D)
  b2 = cb2.reshape(NCONV, 1, D)

  embed104 = jnp.pad(embed, ((0, 4), (0, 0)))
  h = _embed_gather(embed104, atoms2d)[:N]

  msga, msgb = _mlp_first(h, cW1[0], b1[0], cW2[0], b2[0])
  for i in range(1, NCONV):
    pa = _seg_sum(msga, e_src, e_dst)
    pb = _seg_sum(msgb, e_src, e_dst)
    h, msga, msgb = _mlp_layer(h, pa[0, :N], pa[1, :N], pb[0, :N], pb[1, :N],
                               cW1[i], b1[i], cW2[i], b2[i])
  pa = _seg_sum(msga, e_src, e_dst)
  pb = _seg_sum(msgb, e_src, e_dst)

  h_out, apad, colsum, xyzraw = _head(
      h, pa[0, :N], pa[1, :N], pb[0, :N], pb[1, :N],
      Wc1, bc1.reshape(1, D), Wc2, bc2.reshape(1, NCG), xyzp)
  a = apad[:N]

  ga = _seg_sum(apad, e_src, e_dst)
  anorm, cg_adj, cgxyz8 = _finalize(a, ga[0, :N], ga[1, :N], colsum, xyzraw)
  return (h_out, cgxyz8[:, :3], anorm, cg_adj)
